# baseline (device time: 29138 ns/iter reference)
import jax
import jax.numpy as jnp
from jax import lax
from jax.experimental import pallas as pl
from jax.experimental.pallas import tpu as pltpu

N_DEV = 8
S = 1024
H = 128
E = S + 2 * H
HQ = 8
DH = 128
B = 256
C = B + 2 * H
SCALE = 0.08838834764831843


def kernel(x, Wq, K_ext, V_ext, Wo):
    def body(x_hbm, wq_hbm, k_hbm, v_hbm, wo_hbm, out_hbm,
             xv_ref, wqv_ref, wov_ref, kf_ref, vf_ref, octx_ref,
             copy_sems, out_sems, send_sems, recv_sems):
        my = lax.axis_index("i")
        left = lax.rem(my + N_DEV - 1, N_DEV)
        right = lax.rem(my + 1, N_DEV)
        has_left = my > 0
        has_right = my < N_DEV - 1

        cp_x = pltpu.make_async_copy(x_hbm.at[0], xv_ref, copy_sems.at[0])
        cp_wq = pltpu.make_async_copy(wq_hbm, wqv_ref, copy_sems.at[1])
        cp_x.start()
        cp_wq.start()
        cp_k = pltpu.make_async_copy(
            k_hbm.at[0], kf_ref.at[pl.ds(H, S)], copy_sems.at[3])
        cp_v = pltpu.make_async_copy(
            v_hbm.at[0], vf_ref.at[pl.ds(H, S)], copy_sems.at[4])
        cp_wo = pltpu.make_async_copy(wo_hbm, wov_ref, copy_sems.at[2])

        barrier_sem = pltpu.get_barrier_semaphore()

        @pl.when(has_left)
        def _():
            pl.semaphore_signal(
                barrier_sem, inc=1,
                device_id=(left,), device_id_type=pl.DeviceIdType.MESH)

        @pl.when(has_right)
        def _():
            pl.semaphore_signal(
                barrier_sem, inc=1,
                device_id=(right,), device_id_type=pl.DeviceIdType.MESH)

        n_nbrs = has_left.astype(jnp.int32) + has_right.astype(jnp.int32)
        pl.semaphore_wait(barrier_sem, n_nbrs)

        def left_rdmas():
            return (
                pltpu.make_async_remote_copy(
                    src_ref=k_hbm.at[0, pl.ds(0, H)],
                    dst_ref=kf_ref.at[pl.ds(H + S, H)],
                    send_sem=send_sems.at[0], recv_sem=recv_sems.at[0],
                    device_id=(left,), device_id_type=pl.DeviceIdType.MESH),
                pltpu.make_async_remote_copy(
                    src_ref=v_hbm.at[0, pl.ds(0, H)],
                    dst_ref=vf_ref.at[pl.ds(H + S, H)],
                    send_sem=send_sems.at[2], recv_sem=recv_sems.at[2],
                    device_id=(left,), device_id_type=pl.DeviceIdType.MESH),
            )

        def right_rdmas():
            return (
                pltpu.make_async_remote_copy(
                    src_ref=k_hbm.at[0, pl.ds(S - H, H)],
                    dst_ref=kf_ref.at[pl.ds(0, H)],
                    send_sem=send_sems.at[1], recv_sem=recv_sems.at[1],
                    device_id=(right,), device_id_type=pl.DeviceIdType.MESH),
                pltpu.make_async_remote_copy(
                    src_ref=v_hbm.at[0, pl.ds(S - H, H)],
                    dst_ref=vf_ref.at[pl.ds(0, H)],
                    send_sem=send_sems.at[3], recv_sem=recv_sems.at[3],
                    device_id=(right,), device_id_type=pl.DeviceIdType.MESH),
            )

        @pl.when(has_left)
        def _():
            for r in left_rdmas():
                r.start()

        @pl.when(has_right)
        def _():
            for r in right_rdmas():
                r.start()

        @pl.when(jnp.logical_not(has_left))
        def _():
            kf_ref[pl.ds(0, H)] = jnp.zeros((H, HQ, DH), jnp.float32)
            vf_ref[pl.ds(0, H)] = jnp.zeros((H, HQ, DH), jnp.float32)

        @pl.when(jnp.logical_not(has_right))
        def _():
            kf_ref[pl.ds(H + S, H)] = jnp.zeros((H, HQ, DH), jnp.float32)
            vf_ref[pl.ds(H + S, H)] = jnp.zeros((H, HQ, DH), jnp.float32)

        cp_x.wait()
        cp_wq.wait()
        cp_k.start()
        cp_v.start()
        cp_wo.start()
        q = jnp.dot(xv_ref[...], wqv_ref[...],
                    preferred_element_type=jnp.float32)
        q16 = q.astype(jnp.bfloat16)

        i_loc = lax.broadcasted_iota(jnp.int32, (B, C), 0)
        j_loc = lax.broadcasted_iota(jnp.int32, (B, C), 1)
        band = (j_loc >= i_loc) & (j_loc <= i_loc + 2 * H)

        def do_block(b):
            c0 = b * B
            gk0 = my * S - H + c0
            mask = band & (gk0 + j_loc >= 0) & (gk0 + j_loc < N_DEV * S)
            kfb = kf_ref[pl.ds(c0, C)].reshape(C, HQ * DH).astype(jnp.bfloat16)
            vfb = vf_ref[pl.ds(c0, C)].reshape(C, HQ * DH).astype(jnp.bfloat16)
            ctx_parts = []
            for h in range(HQ):
                sl = slice(h * DH, (h + 1) * DH)
                s = lax.dot_general(
                    q16[c0:c0 + B, sl], kfb[:, sl],
                    (((1,), (1,)), ((), ())),
                    preferred_element_type=jnp.float32,
                ) * SCALE
                w = jnp.where(mask, jnp.exp(s), 0.0)
                denom = jnp.sum(w, axis=1, keepdims=True)
                ctx_parts.append(
                    jnp.dot(w.astype(jnp.bfloat16), vfb[:, sl],
                            preferred_element_type=jnp.float32) / denom)
            ctx = jnp.concatenate(ctx_parts, axis=1)
            octx_ref[pl.ds(c0, B)] = jnp.dot(
                ctx, wov_ref[...], preferred_element_type=jnp.float32)
            cp_out = pltpu.make_async_copy(
                octx_ref.at[pl.ds(c0, B)], out_hbm.at[0, pl.ds(c0, B)],
                out_sems.at[b])
            cp_out.start()
            return cp_out

        cp_wo.wait()
        cp_k.wait()
        cp_v.wait()
        outs = [None] * 4
        outs[1] = do_block(1)
        outs[2] = do_block(2)

        @pl.when(has_left)
        def _():
            for r in right_rdmas():
                r.wait_recv()

        outs[0] = do_block(0)

        @pl.when(has_right)
        def _():
            for r in left_rdmas():
                r.wait_recv()

        outs[3] = do_block(3)

        @pl.when(has_left)
        def _():
            for r in left_rdmas():
                r.wait_send()

        @pl.when(has_right)
        def _():
            for r in right_rdmas():
                r.wait_send()

        for cp in outs:
            cp.wait()

    return pl.pallas_call(
        body,
        out_shape=jax.ShapeDtypeStruct((1, S, HQ * DH), jnp.float32),
        in_specs=[pl.BlockSpec(memory_space=pl.ANY)] * 5,
        out_specs=pl.BlockSpec(memory_space=pl.ANY),
        scratch_shapes=[
            pltpu.VMEM((S, HQ * DH), jnp.float32),
            pltpu.VMEM((HQ * DH, HQ * DH), jnp.float32),
            pltpu.VMEM((HQ * DH, HQ * DH), jnp.float32),
            pltpu.VMEM((E, HQ, DH), jnp.float32),
            pltpu.VMEM((E, HQ, DH), jnp.float32),
            pltpu.VMEM((S, HQ * DH), jnp.float32),
            pltpu.SemaphoreType.DMA((5,)),
            pltpu.SemaphoreType.DMA((4,)),
            pltpu.SemaphoreType.DMA((4,)),
            pltpu.SemaphoreType.DMA((4,)),
        ],
        compiler_params=pltpu.CompilerParams(
            collective_id=0,
            vmem_limit_bytes=100 * 1024 * 1024,
        ),
    )(x, Wq, K_ext, V_ext, Wo)


# device time: 25604 ns/iter; 1.1380x vs baseline; 1.1380x over previous
import jax
import jax.numpy as jnp
from jax import lax
from jax.experimental import pallas as pl
from jax.experimental.pallas import tpu as pltpu

N_DEV = 8
S = 1024
H = 128
HD = 1024
E = S + 2 * H
HQ = 8
DH = 128
B = 256
C = B + 2 * H
SCALE = 0.08838834764831843

CHUNKS = ((0, 384), (384, 256), (640, 256), (896, 128))


def kernel(x, Wq, K_ext, V_ext, Wo):
    def body(x_hbm, wq_hbm, k_hbm, v_hbm, wo_hbm, out_hbm,
             xv_ref, wqv_ref, wov_ref, kf_ref, vf_ref, octx_ref,
             hk0_ref, hk1_ref, hv0_ref, hv1_ref,
             sk0_ref, sk1_ref, sv0_ref, sv1_ref,
             rkh_ref, rkt_ref, rvh_ref, rvt_ref,
             copy_sems, out_sems, send_sems, recv_sems):
        my = lax.axis_index("i")
        left = lax.rem(my + N_DEV - 1, N_DEV)
        right = lax.rem(my + 1, N_DEV)
        has_left = my > 0
        has_right = my < N_DEV - 1

        cp_hk0 = pltpu.make_async_copy(
            k_hbm.at[0, pl.ds(0, H)], hk0_ref, copy_sems.at[0])
        cp_hk1 = pltpu.make_async_copy(
            k_hbm.at[0, pl.ds(S - H, H)], hk1_ref, copy_sems.at[1])
        cp_hv0 = pltpu.make_async_copy(
            v_hbm.at[0, pl.ds(0, H)], hv0_ref, copy_sems.at[2])
        cp_hv1 = pltpu.make_async_copy(
            v_hbm.at[0, pl.ds(S - H, H)], hv1_ref, copy_sems.at[3])
        cp_x = pltpu.make_async_copy(x_hbm.at[0], xv_ref, copy_sems.at[4])
        cp_wq = pltpu.make_async_copy(wq_hbm, wqv_ref, copy_sems.at[5])
        cp_wo = pltpu.make_async_copy(wo_hbm, wov_ref, copy_sems.at[6])
        cp_k = []
        cp_v = []
        for ci, (r0, rn) in enumerate(CHUNKS):
            cp_k.append(pltpu.make_async_copy(
                k_hbm.at[0, pl.ds(r0, rn)], kf_ref.at[pl.ds(r0, rn)],
                copy_sems.at[7 + ci]))
            cp_v.append(pltpu.make_async_copy(
                v_hbm.at[0, pl.ds(r0, rn)], vf_ref.at[pl.ds(r0, rn)],
                copy_sems.at[11 + ci]))
        for cp in (cp_hk0, cp_hk1, cp_hv0, cp_hv1, cp_x, cp_wq,
                   cp_k[0], cp_v[0], cp_k[1], cp_v[1], cp_wo,
                   cp_k[2], cp_v[2], cp_k[3], cp_v[3]):
            cp.start()

        barrier_sem = pltpu.get_barrier_semaphore()

        @pl.when(has_left)
        def _():
            pl.semaphore_signal(
                barrier_sem, inc=1,
                device_id=(left,), device_id_type=pl.DeviceIdType.MESH)

        @pl.when(has_right)
        def _():
            pl.semaphore_signal(
                barrier_sem, inc=1,
                device_id=(right,), device_id_type=pl.DeviceIdType.MESH)

        n_nbrs = has_left.astype(jnp.int32) + has_right.astype(jnp.int32)
        pl.semaphore_wait(barrier_sem, n_nbrs)

        cp_hk0.wait()
        cp_hv0.wait()
        sk0_ref[...] = hk0_ref[...].reshape(H, HD).astype(jnp.bfloat16)
        sv0_ref[...] = hv0_ref[...].reshape(H, HD).astype(jnp.bfloat16)
        cp_hk1.wait()
        cp_hv1.wait()
        sk1_ref[...] = hk1_ref[...].reshape(H, HD).astype(jnp.bfloat16)
        sv1_ref[...] = hv1_ref[...].reshape(H, HD).astype(jnp.bfloat16)

        def left_rdmas():
            return (
                pltpu.make_async_remote_copy(
                    src_ref=sk0_ref, dst_ref=rkt_ref,
                    send_sem=send_sems.at[0], recv_sem=recv_sems.at[0],
                    device_id=(left,), device_id_type=pl.DeviceIdType.MESH),
                pltpu.make_async_remote_copy(
                    src_ref=sv0_ref, dst_ref=rvt_ref,
                    send_sem=send_sems.at[2], recv_sem=recv_sems.at[2],
                    device_id=(left,), device_id_type=pl.DeviceIdType.MESH),
            )

        def right_rdmas():
            return (
                pltpu.make_async_remote_copy(
                    src_ref=sk1_ref, dst_ref=rkh_ref,
                    send_sem=send_sems.at[1], recv_sem=recv_sems.at[1],
                    device_id=(right,), device_id_type=pl.DeviceIdType.MESH),
                pltpu.make_async_remote_copy(
                    src_ref=sv1_ref, dst_ref=rvh_ref,
                    send_sem=send_sems.at[3], recv_sem=recv_sems.at[3],
                    device_id=(right,), device_id_type=pl.DeviceIdType.MESH),
            )

        @pl.when(has_left)
        def _():
            for r in left_rdmas():
                r.start()

        @pl.when(has_right)
        def _():
            for r in right_rdmas():
                r.start()

        @pl.when(jnp.logical_not(has_left))
        def _():
            rkh_ref[...] = jnp.zeros((H, HD), jnp.bfloat16)
            rvh_ref[...] = jnp.zeros((H, HD), jnp.bfloat16)

        @pl.when(jnp.logical_not(has_right))
        def _():
            rkt_ref[...] = jnp.zeros((H, HD), jnp.bfloat16)
            rvt_ref[...] = jnp.zeros((H, HD), jnp.bfloat16)

        cp_x.wait()
        cp_wq.wait()
        q = jnp.dot(xv_ref[...], wqv_ref[...],
                    preferred_element_type=jnp.float32)
        q16 = q.astype(jnp.bfloat16)

        i_loc = lax.broadcasted_iota(jnp.int32, (B, C), 0)
        j_loc = lax.broadcasted_iota(jnp.int32, (B, C), 1)
        band = (j_loc >= i_loc) & (j_loc <= i_loc + 2 * H)

        def kv_window(b, f_ref, head_ref, tail_ref):
            lo = b * B - H
            if lo < 0:
                mid = f_ref[pl.ds(0, C - H)].reshape(C - H, HD)
                return jnp.concatenate(
                    [head_ref[...], mid.astype(jnp.bfloat16)], axis=0)
            if lo + C > S:
                mid = f_ref[pl.ds(lo, C - H)].reshape(C - H, HD)
                return jnp.concatenate(
                    [mid.astype(jnp.bfloat16), tail_ref[...]], axis=0)
            return f_ref[pl.ds(lo, C)].reshape(C, HD).astype(jnp.bfloat16)

        def do_block(b):
            c0 = b * B
            gk0 = my * S - H + c0
            mask = band & (gk0 + j_loc >= 0) & (gk0 + j_loc < N_DEV * S)
            kfb = kv_window(b, kf_ref, rkh_ref, rkt_ref)
            vfb = kv_window(b, vf_ref, rvh_ref, rvt_ref)
            ctx_parts = []
            for h in range(HQ):
                sl = slice(h * DH, (h + 1) * DH)
                s = lax.dot_general(
                    q16[c0:c0 + B, sl], kfb[:, sl],
                    (((1,), (1,)), ((), ())),
                    preferred_element_type=jnp.float32,
                ) * SCALE
                w = jnp.where(mask, jnp.exp(s), 0.0)
                denom = jnp.sum(w, axis=1, keepdims=True)
                ctx_parts.append(
                    jnp.dot(w.astype(jnp.bfloat16), vfb[:, sl],
                            preferred_element_type=jnp.float32) / denom)
            ctx = jnp.concatenate(ctx_parts, axis=1)
            octx_ref[pl.ds(c0, B)] = jnp.dot(
                ctx, wov_ref[...], preferred_element_type=jnp.float32)
            cp_out = pltpu.make_async_copy(
                octx_ref.at[pl.ds(c0, B)], out_hbm.at[0, pl.ds(c0, B)],
                out_sems.at[b])
            cp_out.start()
            return cp_out

        cp_wo.wait()
        outs = [None] * 4
        cp_k[0].wait()
        cp_v[0].wait()
        cp_k[1].wait()
        cp_v[1].wait()
        outs[1] = do_block(1)
        cp_k[2].wait()
        cp_v[2].wait()
        outs[2] = do_block(2)

        @pl.when(has_left)
        def _():
            for r in right_rdmas():
                r.wait_recv()

        outs[0] = do_block(0)
        cp_k[3].wait()
        cp_v[3].wait()

        @pl.when(has_right)
        def _():
            for r in left_rdmas():
                r.wait_recv()

        outs[3] = do_block(3)

        @pl.when(has_left)
        def _():
            for r in left_rdmas():
                r.wait_send()

        @pl.when(has_right)
        def _():
            for r in right_rdmas():
                r.wait_send()

        for cp in outs:
            cp.wait()

    return pl.pallas_call(
        body,
        out_shape=jax.ShapeDtypeStruct((1, S, HD), jnp.float32),
        in_specs=[pl.BlockSpec(memory_space=pl.ANY)] * 5,
        out_specs=pl.BlockSpec(memory_space=pl.ANY),
        scratch_shapes=[
            pltpu.VMEM((S, HD), jnp.float32),
            pltpu.VMEM((HD, HD), jnp.float32),
            pltpu.VMEM((HD, HD), jnp.float32),
            pltpu.VMEM((S, HQ, DH), jnp.float32),
            pltpu.VMEM((S, HQ, DH), jnp.float32),
            pltpu.VMEM((S, HD), jnp.float32),
            pltpu.VMEM((H, HQ, DH), jnp.float32),
            pltpu.VMEM((H, HQ, DH), jnp.float32),
            pltpu.VMEM((H, HQ, DH), jnp.float32),
            pltpu.VMEM((H, HQ, DH), jnp.float32),
            pltpu.VMEM((H, HD), jnp.bfloat16),
            pltpu.VMEM((H, HD), jnp.bfloat16),
            pltpu.VMEM((H, HD), jnp.bfloat16),
            pltpu.VMEM((H, HD), jnp.bfloat16),
            pltpu.VMEM((H, HD), jnp.bfloat16),
            pltpu.VMEM((H, HD), jnp.bfloat16),
            pltpu.VMEM((H, HD), jnp.bfloat16),
            pltpu.VMEM((H, HD), jnp.bfloat16),
            pltpu.SemaphoreType.DMA((15,)),
            pltpu.SemaphoreType.DMA((4,)),
            pltpu.SemaphoreType.DMA((4,)),
            pltpu.SemaphoreType.DMA((4,)),
        ],
        compiler_params=pltpu.CompilerParams(
            collective_id=0,
            vmem_limit_bytes=100 * 1024 * 1024,
        ),
    )(x, Wq, K_ext, V_ext, Wo)
